# Initial kernel scaffold; baseline (speedup 1.0000x reference)
import jax
import jax.numpy as jnp
from jax.experimental import pallas as pl


def kernel(x, lut):
    raise NotImplementedError("write your pallas kernel here")


# speedup vs baseline: 2.9699x; 2.9699x over previous
import functools

import jax
import jax.numpy as jnp
from jax import lax
from jax.experimental import pallas as pl
from jax.experimental.pallas import tpu as pltpu
from jax.experimental.pallas import tpu_sc as plsc

N_TABLE = 100000
D = 128
B_TOTAL = 4096 * 50
NC, NS = 2, 16
NW = NC * NS
PER_W = B_TOTAL // NW
CHUNK = 128
NCHUNK = PER_W // CHUNK

_MESH = plsc.VectorSubcoreMesh(
    core_axis_name="c", subcore_axis_name="s", num_cores=NC, num_subcores=NS
)


@functools.partial(
    pl.kernel,
    out_type=jax.ShapeDtypeStruct((B_TOTAL, D), jnp.float32),
    mesh=_MESH,
    scratch_types=[
        pltpu.VMEM((PER_W,), jnp.int32),
        pltpu.VMEM((CHUNK, D), jnp.float32),
        pltpu.SemaphoreType.DMA,
    ],
)
def _sc_gather(lut_hbm, idx_hbm, out_hbm, idx_v, rows_v, sem):
    wid = lax.axis_index("s") * NC + lax.axis_index("c")
    base = wid * PER_W
    pltpu.sync_copy(idx_hbm.at[pl.ds(base, PER_W)], idx_v)

    def chunk_body(j, carry):
        idx_slice = idx_v.at[pl.ds(j * CHUNK, CHUNK)]
        pltpu.async_copy(lut_hbm.at[idx_slice], rows_v, sem).wait()
        pltpu.sync_copy(rows_v, out_hbm.at[pl.ds(base + j * CHUNK, CHUNK)])
        return carry

    lax.fori_loop(0, NCHUNK, chunk_body, 0)


def kernel(x, lut):
    idx = jnp.reshape(x.astype(jnp.int32), (B_TOTAL,))
    out = _sc_gather(lut, idx)
    return jnp.reshape(out, (*x.shape, D))


# speedup vs baseline: 3.3258x; 1.1198x over previous
import functools

import jax
import jax.numpy as jnp
from jax import lax
from jax.experimental import pallas as pl
from jax.experimental.pallas import tpu as pltpu
from jax.experimental.pallas import tpu_sc as plsc

N_TABLE = 100000
D = 128
B_TOTAL = 4096 * 50
NC, NS = 2, 16
NW = NC * NS
PER_W = B_TOTAL // NW
CHUNK = 128
NCHUNK = PER_W // CHUNK
NBUF = 5
NGROUP = NCHUNK // NBUF

_MESH = plsc.VectorSubcoreMesh(
    core_axis_name="c", subcore_axis_name="s", num_cores=NC, num_subcores=NS
)


@functools.partial(
    pl.kernel,
    out_type=jax.ShapeDtypeStruct((B_TOTAL, D), jnp.float32),
    mesh=_MESH,
    scratch_types=[
        pltpu.VMEM((PER_W,), jnp.int32),
        pltpu.VMEM((NBUF, CHUNK, D), jnp.float32),
        pltpu.SemaphoreType.DMA((NBUF,)),
        pltpu.SemaphoreType.DMA((NBUF,)),
    ],
)
def _sc_gather(lut_hbm, idx_hbm, out_hbm, idx_v, rows_v, gsem, ssem):
    wid = lax.axis_index("s") * NC + lax.axis_index("c")
    base = wid * PER_W
    pltpu.sync_copy(idx_hbm.at[pl.ds(base, PER_W)], idx_v)

    def start_gather(j, b):
        idx_slice = idx_v.at[pl.ds(j * CHUNK, CHUNK)]
        pltpu.async_copy(lut_hbm.at[idx_slice], rows_v.at[b], gsem.at[b])

    def wait_gather(b):
        idx_slice = idx_v.at[pl.ds(0, CHUNK)]
        pltpu.make_async_copy(lut_hbm.at[idx_slice], rows_v.at[b], gsem.at[b]).wait()

    def start_store(j, b):
        pltpu.async_copy(
            rows_v.at[b], out_hbm.at[pl.ds(base + j * CHUNK, CHUNK)], ssem.at[b]
        )

    def wait_store(b):
        pltpu.make_async_copy(
            rows_v.at[b], out_hbm.at[pl.ds(base, CHUNK)], ssem.at[b]
        ).wait()

    for b in range(NBUF - 1):
        start_gather(b, b)

    for b in range(NBUF):
        wait_gather(b)
        start_store(b, b)
        if b > 0:
            wait_store(b - 1)
        start_gather(b + NBUF - 1, (b - 1) % NBUF)

    def group_body(g, carry):
        j0 = g * NBUF
        for b in range(NBUF):
            j = j0 + b
            wait_gather(b)
            start_store(j, b)
            bb = (b - 1) % NBUF
            wait_store(bb)
            start_gather(j + NBUF - 1, bb)
        return carry

    lax.fori_loop(1, NGROUP - 1, group_body, 0)

    j0 = NCHUNK - NBUF
    wait_gather(0)
    start_store(j0, 0)
    wait_store(NBUF - 1)
    start_gather(j0 + NBUF - 1, NBUF - 1)
    for b in range(1, NBUF):
        wait_gather(b)
        start_store(j0 + b, b)
    for b in range(NBUF):
        wait_store(b)


def kernel(x, lut):
    idx = jnp.reshape(x.astype(jnp.int32), (B_TOTAL,))
    out = _sc_gather(lut, idx)
    return jnp.reshape(out, (*x.shape, D))
